# baseline (speedup 1.0000x reference)
import jax
import jax.numpy as jnp
from jax import lax
from jax.experimental import pallas as pl
from jax.experimental.pallas import tpu as pltpu
from jax.experimental.pallas import tpu_sc as plsc

VOCAB = 1000
EMBED = 128
BATCH = 16384
HIST = 200

NC = 2
NS = 16
NW = NC * NS

B = BATCH * HIST
B_PER_W = B // NW
CHUNK = 128
NBUF = 4
LEAD = 1
IDX_BLOCK = 12800
N_INNER = IDX_BLOCK // CHUNK
N_OUTER = B_PER_W // IDX_BLOCK


def _body(idx_hbm, table_hbm, out_hbm, idx_v, table_sp, r0, r1, r2, r3,
          g0, g1, g2, g3, s0, s1, s2, s3):
    rows = [r0, r1, r2, r3]
    gsem = [g0, g1, g2, g3]
    ssem = [s0, s1, s2, s3]

    sid = lax.axis_index("s")
    wid = sid * NC + lax.axis_index("c")
    base = wid * B_PER_W

    @pl.when(sid == 0)
    def _():
        pltpu.sync_copy(table_hbm, table_sp)

    plsc.subcore_barrier()

    def gather_start(c, b):
        pltpu.async_copy(
            table_sp.at[idx_v.at[pl.ds(c * CHUNK, CHUNK)]], rows[b], gsem[b]
        )

    def gather_wait(b):
        pltpu.make_async_copy(
            table_sp.at[pl.ds(0, CHUNK)], rows[b], gsem[b]
        ).wait()

    def store_start(blk, c, b):
        pltpu.async_copy(
            rows[b], out_hbm.at[pl.ds(blk + c * CHUNK, CHUNK)], ssem[b]
        )

    def store_wait(blk, b):
        pltpu.make_async_copy(
            rows[b], out_hbm.at[pl.ds(blk, CHUNK)], ssem[b]
        ).wait()

    def outer(ib, _):
        blk = base + ib * IDX_BLOCK
        pltpu.sync_copy(idx_hbm.at[pl.ds(blk, IDX_BLOCK)], idx_v)

        for b in range(LEAD):
            gather_start(b, b)

        def inner(o, _):
            for b in range(NBUF):
                c = o * NBUF + b
                gather_wait(b)
                store_start(blk, c, b)
                nb = (b + LEAD) % NBUF
                cc = c + LEAD

                @pl.when(cc < N_INNER)
                def _():
                    @pl.when(c >= NBUF - LEAD)
                    def _():
                        store_wait(blk, nb)

                    gather_start(cc, nb)
            return 0

        lax.fori_loop(0, N_INNER // NBUF, inner, 0)
        for b in range(NBUF):
            store_wait(blk, b)
        return 0

    lax.fori_loop(0, N_OUTER, outer, 0)


@jax.jit
def _embed(idx_flat, table):
    mesh = plsc.VectorSubcoreMesh(core_axis_name="c", subcore_axis_name="s")
    run = pl.kernel(
        _body,
        out_type=jax.ShapeDtypeStruct((B, EMBED), jnp.float32),
        mesh=mesh,
        scratch_types=(
            [pltpu.VMEM((IDX_BLOCK,), jnp.int32),
             pltpu.VMEM_SHARED((VOCAB, EMBED), jnp.float32)]
            + [pltpu.VMEM((CHUNK, EMBED), jnp.float32) for _ in range(NBUF)]
            + [pltpu.SemaphoreType.DMA for _ in range(2 * NBUF)]
        ),
    )
    return run(idx_flat, table)


def kernel(visit_segments, table):
    idx_flat = visit_segments.reshape(B)
    out = _embed(idx_flat, table)
    return out.reshape(BATCH, HIST, EMBED)


# speedup vs baseline: 1.0593x; 1.0593x over previous
import jax
import jax.numpy as jnp
from jax import lax
from jax.experimental import pallas as pl
from jax.experimental.pallas import tpu as pltpu
from jax.experimental.pallas import tpu_sc as plsc

VOCAB = 1000
EMBED = 128
BATCH = 16384
HIST = 200

NC = 2
NS = 16
NW = NC * NS

B = BATCH * HIST
B_PER_W = B // NW
CHUNK = 128
NBUF = 4
LEAD = 2
IDX_BLOCK = 12800
N_INNER = IDX_BLOCK // CHUNK
N_OUTER = B_PER_W // IDX_BLOCK


def _body(idx_hbm, table_hbm, out_hbm, idx_v, table_sp, r0, r1, r2, r3,
          g0, g1, g2, g3, s0, s1, s2, s3):
    rows = [r0, r1, r2, r3]
    gsem = [g0, g1, g2, g3]
    ssem = [s0, s1, s2, s3]

    sid = lax.axis_index("s")
    wid = sid * NC + lax.axis_index("c")
    base = wid * B_PER_W

    @pl.when(sid == 0)
    def _():
        pltpu.sync_copy(table_hbm, table_sp)

    plsc.subcore_barrier()

    def gather_start(c, b):
        pltpu.async_copy(
            table_sp.at[idx_v.at[pl.ds(c * CHUNK, CHUNK)]], rows[b], gsem[b]
        )

    def gather_wait(b):
        pltpu.make_async_copy(
            table_sp.at[pl.ds(0, CHUNK)], rows[b], gsem[b]
        ).wait()

    def store_start(blk, c, b):
        pltpu.async_copy(
            rows[b], out_hbm.at[pl.ds(blk + c * CHUNK, CHUNK)], ssem[b]
        )

    def store_wait(blk, b):
        pltpu.make_async_copy(
            rows[b], out_hbm.at[pl.ds(blk, CHUNK)], ssem[b]
        ).wait()

    def outer(ib, _):
        blk = base + ib * IDX_BLOCK
        pltpu.sync_copy(idx_hbm.at[pl.ds(blk, IDX_BLOCK)], idx_v)

        for b in range(LEAD):
            gather_start(b, b)

        def inner(o, _):
            for b in range(NBUF):
                c = o * NBUF + b
                gather_wait(b)
                store_start(blk, c, b)
                nb = (b + LEAD) % NBUF
                cc = c + LEAD

                @pl.when(cc < N_INNER)
                def _():
                    @pl.when(c >= NBUF - LEAD)
                    def _():
                        store_wait(blk, nb)

                    gather_start(cc, nb)
            return 0

        lax.fori_loop(0, N_INNER // NBUF, inner, 0)
        for b in range(NBUF):
            store_wait(blk, b)
        return 0

    lax.fori_loop(0, N_OUTER, outer, 0)


@jax.jit
def _embed(idx_flat, table):
    mesh = plsc.VectorSubcoreMesh(core_axis_name="c", subcore_axis_name="s")
    run = pl.kernel(
        _body,
        out_type=jax.ShapeDtypeStruct((B, EMBED), jnp.float32),
        mesh=mesh,
        scratch_types=(
            [pltpu.VMEM((IDX_BLOCK,), jnp.int32),
             pltpu.VMEM_SHARED((VOCAB, EMBED), jnp.float32)]
            + [pltpu.VMEM((CHUNK, EMBED), jnp.float32) for _ in range(NBUF)]
            + [pltpu.SemaphoreType.DMA for _ in range(2 * NBUF)]
        ),
    )
    return run(idx_flat, table)


def kernel(visit_segments, table):
    idx_flat = visit_segments.reshape(B)
    out = _embed(idx_flat, table)
    return out.reshape(BATCH, HIST, EMBED)
